# Initial kernel scaffold; baseline (speedup 1.0000x reference)
import jax
import jax.numpy as jnp
from jax.experimental import pallas as pl


def kernel(user_emb, entity_emb, latent_emb, relation_emb, disen_weight_att, interact_values, edge_index, edge_type, interact_row, interact_col):
    raise NotImplementedError("write your pallas kernel here")


# speedup vs baseline: 2.0829x; 2.0829x over previous
import functools

import jax
import jax.numpy as jnp
from jax import lax
from jax.experimental import pallas as pl
from jax.experimental.pallas import tpu as pltpu
from jax.experimental.pallas import tpu_sc as plsc

N_ENT = 10000
N_USR = 20000
N_ENT_P = 10240
N_USR_P = 20480
D = 128
HD = 64
N_REL = 16
N_EDGE = 320000
NNZ = 500000
C = 80

KG_CHUNKS_PER_TILE = N_EDGE // C // 16
U_CHUNKS = NNZ // C
U_CHUNKS_PER_TILE = -(-U_CHUNKS // 16)

_ENT_SLICE = N_ENT_P // 16
_USR_SLICE = N_USR_P // 16
_ZROWS = 160


def _fill_zero_2d(ref, rows):
    def body(i, _):
        for j in range(HD // 16):
            ref[i, pl.ds(j * 16, 16)] = jnp.zeros((16,), jnp.float32)
        return 0
    lax.fori_loop(0, rows, body, 0)


def _fill_const_1d(ref, n16, val):
    def body(i, _):
        ref[pl.ds(i * 16, 16)] = jnp.full((16,), val, jnp.float32)
        return 0
    lax.fori_loop(0, n16, body, 0)


def _make_sc_kernel(with_cnt: bool):
    mesh = plsc.VectorSubcoreMesh(core_axis_name="c", subcore_axis_name="s",
                                  num_cores=2, num_subcores=16)

    out_type = [
        jax.ShapeDtypeStruct((N_ENT_P, HD), jnp.float32),
        jax.ShapeDtypeStruct((N_ENT_P, HD), jnp.float32),
        jax.ShapeDtypeStruct((N_USR_P, HD), jnp.float32),
        jax.ShapeDtypeStruct((N_USR_P, HD), jnp.float32),
    ]
    if with_cnt:
        out_type.append(jax.ShapeDtypeStruct((N_ENT_P,), jnp.float32))

    scratch = dict(
        acc=pltpu.VMEM_SHARED((N_USR_P, HD), jnp.float32),
        t_idx=pltpu.VMEM((C,), jnp.int32),
        r_idx=pltpu.VMEM((C,), jnp.int32),
        h_idx=pltpu.VMEM((C,), jnp.int32),
        v_buf=pltpu.VMEM((C,), jnp.float32),
        rows=pltpu.VMEM((C, HD), jnp.float32),
        rel_loc=pltpu.VMEM((2 * N_REL, HD), jnp.float32),
        zbuf=pltpu.VMEM((_ZROWS, HD), jnp.float32),
        ones=pltpu.VMEM((C,), jnp.float32),
        sem=pltpu.SemaphoreType.DMA,
    )
    if with_cnt:
        scratch["acc_c"] = pltpu.VMEM_SHARED((N_ENT_P,), jnp.float32)
        scratch["zbuf1"] = pltpu.VMEM((640,), jnp.float32)

    def body(ent2, rel2, tail2, etype, head, col2, urow, uval, *refs,
             acc, t_idx, r_idx, h_idx, v_buf, rows, rel_loc, zbuf, ones, sem,
             acc_c=None, zbuf1=None):
        if with_cnt:
            s0, s1, u0, u1, cnt_out = refs
        else:
            s0, s1, u0, u1 = refs

        c = lax.axis_index("c")
        s = lax.axis_index("s")

        pltpu.sync_copy(rel2, rel_loc)
        _fill_zero_2d(zbuf, _ZROWS)
        _fill_const_1d(ones, C // 16, 1.0)
        if with_cnt:
            _fill_const_1d(zbuf1, 40, 0.0)

        for b in range(_ENT_SLICE // _ZROWS):
            pltpu.sync_copy(
                zbuf, acc.at[pl.ds(s * _ENT_SLICE + b * _ZROWS, _ZROWS), :])
        if with_cnt:
            @pl.when(c == 0)
            def _():
                pltpu.sync_copy(zbuf1, acc_c.at[pl.ds(s * 640, 640)])
        plsc.subcore_barrier()

        def kg_body(k, _):
            base = (s * KG_CHUNKS_PER_TILE + k) * C
            pltpu.sync_copy(tail2.at[pl.ds(base, C)], t_idx)
            pltpu.sync_copy(etype.at[pl.ds(base, C)], r_idx)
            pltpu.sync_copy(head.at[pl.ds(base, C)], h_idx)
            for j in range(C // 16):
                t_idx[pl.ds(j * 16, 16)] = t_idx[pl.ds(j * 16, 16)] + c
                r_idx[pl.ds(j * 16, 16)] = r_idx[pl.ds(j * 16, 16)] * 2 + c
            pltpu.async_copy(ent2.at[t_idx], rows, sem).wait()

            def mul_body(g, _):
                tv = r_idx[pl.ds(g * 16, 16)]
                for r2 in range(16):
                    t = tv[r2]
                    row = g * 16 + r2
                    for j in range(HD // 16):
                        ds = pl.ds(j * 16, 16)
                        rows[row, ds] = rows[row, ds] * rel_loc[t, ds]
                return 0
            lax.fori_loop(0, C // 16, mul_body, 0)

            pltpu.sync_copy(rows, acc.at[h_idx], add=True)
            if with_cnt:
                @pl.when(c == 0)
                def _():
                    pltpu.sync_copy(ones, acc_c.at[h_idx], add=True)
            return 0
        lax.fori_loop(0, KG_CHUNKS_PER_TILE, kg_body, 0)
        plsc.subcore_barrier()

        sl = pl.ds(s * _ENT_SLICE, _ENT_SLICE)

        @pl.when(c == 0)
        def _():
            pltpu.sync_copy(acc.at[sl, :], s0.at[sl, :])

        @pl.when(c == 1)
        def _():
            pltpu.sync_copy(acc.at[sl, :], s1.at[sl, :])
        if with_cnt:
            @pl.when(c == 0)
            def _():
                pltpu.sync_copy(acc_c.at[pl.ds(s * 640, 640)],
                                cnt_out.at[pl.ds(s * 640, 640)])
        plsc.subcore_barrier()

        for b in range(_USR_SLICE // _ZROWS):
            pltpu.sync_copy(
                zbuf, acc.at[pl.ds(s * _USR_SLICE + b * _ZROWS, _ZROWS), :])
        plsc.subcore_barrier()

        def u_body(k, _):
            g = s + 16 * k

            @pl.when(g < U_CHUNKS)
            def _():
                base = g * C
                pltpu.sync_copy(col2.at[pl.ds(base, C)], t_idx)
                pltpu.sync_copy(urow.at[pl.ds(base, C)], h_idx)
                pltpu.sync_copy(uval.at[pl.ds(base, C)], v_buf)
                for j in range(C // 16):
                    t_idx[pl.ds(j * 16, 16)] = t_idx[pl.ds(j * 16, 16)] + c
                pltpu.async_copy(ent2.at[t_idx], rows, sem).wait()

                def scale_body(g2, _):
                    vv = v_buf[pl.ds(g2 * 16, 16)]
                    for r2 in range(16):
                        sv = vv[r2]
                        row = g2 * 16 + r2
                        for j in range(HD // 16):
                            ds = pl.ds(j * 16, 16)
                            rows[row, ds] = rows[row, ds] * sv
                    return 0
                lax.fori_loop(0, C // 16, scale_body, 0)

                pltpu.sync_copy(rows, acc.at[h_idx], add=True)
            return 0
        lax.fori_loop(0, U_CHUNKS_PER_TILE, u_body, 0)
        plsc.subcore_barrier()

        for b in range(_USR_SLICE // _ZROWS):
            slb = pl.ds(s * _USR_SLICE + b * _ZROWS, _ZROWS)

            @pl.when(c == 0)
            def _():
                pltpu.sync_copy(acc.at[slb, :], u0.at[slb, :])

            @pl.when(c == 1)
            def _():
                pltpu.sync_copy(acc.at[slb, :], u1.at[slb, :])

    return pl.kernel(body, out_type=tuple(out_type), mesh=mesh,
                     scratch_types=scratch,
                     compiler_params=pltpu.CompilerParams(
                         use_tc_tiling_on_sc=False))


_sc_cache = {}


def _sc_kernel(with_cnt: bool):
    if with_cnt not in _sc_cache:
        _sc_cache[with_cnt] = _make_sc_kernel(with_cnt)
    return _sc_cache[with_cnt]


_BE = 1000


def _entity_body(s0, s1, cnt, res, enew, rout):
    sfull = jnp.concatenate([s0[...], s1[...]], axis=1)
    cv = jnp.maximum(cnt[...], 1.0)
    agg = sfull / cv
    nrm = jnp.sqrt(jnp.sum(agg * agg, axis=1, keepdims=True))
    e = agg / jnp.maximum(nrm, 1e-12)
    enew[...] = e
    rout[...] = res[...] + e


def _tc_entity(s0, s1, cnt, res_in):
    n = res_in.shape[0]
    grid = (n // _BE,)
    return pl.pallas_call(
        _entity_body,
        grid=grid,
        in_specs=[
            pl.BlockSpec((_BE, HD), lambda i: (i, 0)),
            pl.BlockSpec((_BE, HD), lambda i: (i, 0)),
            pl.BlockSpec((_BE, 1), lambda i: (i, 0)),
            pl.BlockSpec((_BE, D), lambda i: (i, 0)),
        ],
        out_specs=[
            pl.BlockSpec((_BE, D), lambda i: (i, 0)),
            pl.BlockSpec((_BE, D), lambda i: (i, 0)),
        ],
        out_shape=[
            jax.ShapeDtypeStruct((n, D), jnp.float32),
            jax.ShapeDtypeStruct((n, D), jnp.float32),
        ],
    )(s0, s1, cnt, res_in)


def _user_body(u0, u1, uemb, latr, dwr, res, unew, rout):
    ua = jnp.concatenate([u0[...], u1[...]], axis=1)
    logits = lax.dot_general(
        uemb[...], latr[...], (((1,), (1,)), ((), ())),
        preferred_element_type=jnp.float32, precision=lax.Precision.HIGHEST)
    m = jnp.max(logits, axis=1, keepdims=True)
    p = jnp.exp(logits - m)
    p = p / jnp.sum(p, axis=1, keepdims=True)
    factor = lax.dot_general(
        p, dwr[...], (((1,), (0,)), ((), ())),
        preferred_element_type=jnp.float32, precision=lax.Precision.HIGHEST)
    out = factor * ua + ua
    nrm = jnp.sqrt(jnp.sum(out * out, axis=1, keepdims=True))
    u = out / jnp.maximum(nrm, 1e-12)
    unew[...] = u
    rout[...] = res[...] + u


def _tc_user(u0, u1, uemb, latent, dw, res_in):
    n = uemb.shape[0]
    grid = (n // _BE,)
    nf = latent.shape[0]
    return pl.pallas_call(
        _user_body,
        grid=grid,
        in_specs=[
            pl.BlockSpec((_BE, HD), lambda i: (i, 0)),
            pl.BlockSpec((_BE, HD), lambda i: (i, 0)),
            pl.BlockSpec((_BE, D), lambda i: (i, 0)),
            pl.BlockSpec((nf, D), lambda i: (0, 0)),
            pl.BlockSpec((nf, D), lambda i: (0, 0)),
            pl.BlockSpec((_BE, D), lambda i: (i, 0)),
        ],
        out_specs=[
            pl.BlockSpec((_BE, D), lambda i: (i, 0)),
            pl.BlockSpec((_BE, D), lambda i: (i, 0)),
        ],
        out_shape=[
            jax.ShapeDtypeStruct((n, D), jnp.float32),
            jax.ShapeDtypeStruct((n, D), jnp.float32),
        ],
    )(u0, u1, uemb, latent, dw, res_in)


def _prep_body(att_ref, rel_ref, dw_ref, cor_ref):
    att = att_ref[...]
    m = jnp.max(att, axis=1, keepdims=True)
    p = jnp.exp(att - m)
    p = p / jnp.sum(p, axis=1, keepdims=True)
    dw_ref[...] = lax.dot_general(
        p, rel_ref[...], (((1,), (0,)), ((), ())),
        preferred_element_type=jnp.float32, precision=lax.Precision.HIGHEST)
    nrm = jnp.sqrt(jnp.sum(att * att, axis=1, keepdims=True))
    nt = att / jnp.maximum(nrm, 1e-12)
    sim = lax.dot_general(
        nt, nt, (((1,), (1,)), ((), ())),
        preferred_element_type=jnp.float32, precision=lax.Precision.HIGHEST)
    sc = jnp.exp(sim / 0.2)
    rows = jnp.sum(sc, axis=1)
    nf = att.shape[0]
    ii = lax.broadcasted_iota(jnp.int32, (nf, nf), 0)
    jj = lax.broadcasted_iota(jnp.int32, (nf, nf), 1)
    diag = jnp.sum(jnp.where(ii == jj, sc, 0.0), axis=1)
    cor_ref[...] = (-jnp.sum(jnp.log(diag) - jnp.log(rows))).reshape(1, 1)


def _tc_prep(att, rel):
    nf, nr = att.shape
    return pl.pallas_call(
        _prep_body,
        out_shape=[
            jax.ShapeDtypeStruct((nf, D), jnp.float32),
            jax.ShapeDtypeStruct((1, 1), jnp.float32),
        ],
    )(att, rel)


def kernel(user_emb, entity_emb, latent_emb, relation_emb, disen_weight_att,
           interact_values, edge_index, edge_type, interact_row, interact_col):
    head = edge_index[0]
    tail2 = edge_index[1] * 2
    col2 = interact_col * 2
    rel2 = relation_emb.reshape(2 * N_REL, HD)

    ent2 = entity_emb.reshape(2 * N_ENT, HD)
    s0, s1, u0, u1, cnt = _sc_kernel(True)(ent2, rel2, tail2, edge_type, head,
                                           col2, interact_row, interact_values)
    cnt2d = cnt.reshape(N_ENT_P, 1)
    e1, eres1 = _tc_entity(s0, s1, cnt2d, entity_emb)
    dw, cor = _tc_prep(disen_weight_att, relation_emb)
    u1n, ures1 = _tc_user(u0, u1, user_emb, latent_emb, dw, user_emb)

    ent2b = e1.reshape(2 * N_ENT, HD)
    s0b, s1b, u0b, u1b = _sc_kernel(False)(ent2b, rel2, tail2, edge_type, head,
                                           col2, interact_row, interact_values)
    e2, eres2 = _tc_entity(s0b, s1b, cnt2d, eres1)
    u2n, ures2 = _tc_user(u0b, u1b, u1n, latent_emb, dw, ures1)

    return (eres2, ures2, cor.reshape(()))


# speedup vs baseline: 3.4803x; 1.6709x over previous
import functools

import jax
import jax.numpy as jnp
from jax import lax
from jax.experimental import pallas as pl
from jax.experimental.pallas import tpu as pltpu
from jax.experimental.pallas import tpu_sc as plsc

N_ENT = 10000
N_USR = 20000
N_ENT_P = 10240
N_USR_P = 20480
D = 128
HD = 64
N_REL = 16
N_EDGE = 320000
NNZ = 500000
C = 80
SUB = 10

KG_SUPERS = N_EDGE // (C * SUB)
KG_SUPERS_PER_TILE = KG_SUPERS // 16
U_SUPERS = NNZ // (C * SUB)
U_SUPERS_PER_TILE = -(-U_SUPERS // 16)

_ENT_SLICE = N_ENT_P // 16
_USR_SLICE = N_USR_P // 16
_ZROWS = 160


def _fill_zero_2d(ref, rows):
    def body(i, _):
        for j in range(HD // 16):
            ref[i, pl.ds(j * 16, 16)] = jnp.zeros((16,), jnp.float32)
        return 0
    lax.fori_loop(0, rows, body, 0)


def _fill_const_1d(ref, n16, val):
    def body(i, _):
        ref[pl.ds(i * 16, 16)] = jnp.full((16,), val, jnp.float32)
        return 0
    lax.fori_loop(0, n16, body, 0)


def _make_sc_kernel():
    mesh = plsc.VectorSubcoreMesh(core_axis_name="c", subcore_axis_name="s",
                                  num_cores=2, num_subcores=16)

    out_type = [
        jax.ShapeDtypeStruct((N_ENT_P, HD), jnp.float32),
        jax.ShapeDtypeStruct((N_ENT_P, HD), jnp.float32),
        jax.ShapeDtypeStruct((N_USR_P, HD), jnp.float32),
        jax.ShapeDtypeStruct((N_USR_P, HD), jnp.float32),
        jax.ShapeDtypeStruct((N_ENT_P,), jnp.float32),
    ]

    scratch = dict(
        acc=pltpu.VMEM_SHARED((N_USR_P, HD), jnp.float32),
        t_idx=pltpu.VMEM((SUB, C), jnp.int32),
        r_idx=pltpu.VMEM((SUB, C), jnp.int32),
        h_idx=pltpu.VMEM((SUB, C), jnp.int32),
        v_buf=pltpu.VMEM((SUB, C), jnp.float32),
        rows0=pltpu.VMEM((C, HD), jnp.float32),
        rows1=pltpu.VMEM((C, HD), jnp.float32),
        rel_loc=pltpu.VMEM((N_REL, HD), jnp.float32),
        zbuf=pltpu.VMEM((_ZROWS, HD), jnp.float32),
        ones=pltpu.VMEM((C,), jnp.float32),
        sem_i=pltpu.SemaphoreType.DMA,
        sem_g=pltpu.SemaphoreType.DMA,
        sem_s0=pltpu.SemaphoreType.DMA,
        sem_s1=pltpu.SemaphoreType.DMA,
        acc_c=pltpu.VMEM_SHARED((N_ENT_P,), jnp.float32),
        zbuf1=pltpu.VMEM((640,), jnp.float32),
    )

    def body(entL, entR, relL, relR, tailr, typer, headr, colr, urowr, uvalr,
             *refs, acc, t_idx, r_idx, h_idx, v_buf, rows0, rows1, rel_loc,
             zbuf, ones, sem_i, sem_g, sem_s0, sem_s1, acc_c, zbuf1):
        s0, s1, u0, u1, cnt_out = refs

        c = lax.axis_index("c")
        s = lax.axis_index("s")

        @pl.when(c == 0)
        def _():
            pltpu.sync_copy(relL, rel_loc)

        @pl.when(c == 1)
        def _():
            pltpu.sync_copy(relR, rel_loc)
        _fill_zero_2d(zbuf, _ZROWS)
        _fill_const_1d(ones, C // 16, 1.0)
        _fill_const_1d(zbuf1, 40, 0.0)

        def ent_gather_start(idx_row, buf):
            @pl.when(c == 0)
            def _():
                pltpu.async_copy(entL.at[idx_row], buf, sem_g)

            @pl.when(c == 1)
            def _():
                pltpu.async_copy(entR.at[idx_row], buf, sem_g)

        def run_phase(kg):
            nsup = KG_SUPERS_PER_TILE if kg else U_SUPERS_PER_TILE
            iarr = tailr if kg else colr
            harr = headr if kg else urowr
            bufs = (rows0, rows1)
            ssem = (sem_s0, sem_s1)

            def gather_start(j, buf):
                ent_gather_start(t_idx.at[j], buf)

            def gather_wait(buf):
                pltpu.make_async_copy(entL.at[t_idx.at[0]], buf, sem_g).wait()

            def scatter_wait(j):
                pltpu.make_async_copy(bufs[j % 2], acc.at[h_idx.at[0]],
                                      ssem[j % 2]).wait()

            def mul_chunk(j, buf):
                def mul_body(g, _):
                    tv = r_idx[j, pl.ds(g * 16, 16)]
                    for r2 in range(16):
                        t = tv[r2]
                        row = g * 16 + r2
                        for jj in range(HD // 16):
                            ds = pl.ds(jj * 16, 16)
                            buf[row, ds] = buf[row, ds] * rel_loc[t, ds]
                    return 0
                lax.fori_loop(0, C // 16, mul_body, 0)

            def scale_chunk(j, buf):
                def scale_body(g, _):
                    vv = v_buf[j, pl.ds(g * 16, 16)]
                    for r2 in range(16):
                        sv = vv[r2]
                        row = g * 16 + r2
                        for jj in range(HD // 16):
                            ds = pl.ds(jj * 16, 16)
                            buf[row, ds] = buf[row, ds] * sv
                    return 0
                lax.fori_loop(0, C // 16, scale_body, 0)

            def run_super(m):
                sid = (s * KG_SUPERS_PER_TILE + m) if kg else (s + 16 * m)
                pltpu.async_copy(iarr.at[sid], t_idx, sem_i)
                pltpu.async_copy(harr.at[sid], h_idx, sem_i)
                if kg:
                    pltpu.async_copy(typer.at[sid], r_idx, sem_i)
                else:
                    pltpu.async_copy(uvalr.at[sid], v_buf, sem_i)
                pltpu.make_async_copy(iarr.at[0], t_idx, sem_i).wait()
                pltpu.make_async_copy(harr.at[0], h_idx, sem_i).wait()
                if kg:
                    pltpu.make_async_copy(typer.at[0], r_idx, sem_i).wait()
                else:
                    pltpu.make_async_copy(uvalr.at[0], v_buf, sem_i).wait()

                gather_start(0, bufs[0])
                for j in range(SUB):
                    b = j % 2
                    gather_wait(bufs[b])
                    if j > 0:
                        scatter_wait(j - 1)
                    if j < SUB - 1:
                        gather_start(j + 1, bufs[1 - b])
                    if kg:
                        mul_chunk(j, bufs[b])
                    else:
                        scale_chunk(j, bufs[b])
                    pltpu.async_copy(bufs[b], acc.at[h_idx.at[j]], ssem[b],
                                     add=True)
                    if kg:
                        @pl.when(c == 0)
                        def _():
                            pltpu.sync_copy(ones, acc_c.at[h_idx.at[j]],
                                            add=True)
                scatter_wait(SUB - 1)

            def sup_body(m, _):
                if kg:
                    run_super(m)
                else:
                    @pl.when(s + 16 * m < U_SUPERS)
                    def _():
                        run_super(m)
                return 0
            lax.fori_loop(0, nsup, sup_body, 0)

        for bb in range(_ENT_SLICE // _ZROWS):
            pltpu.sync_copy(
                zbuf, acc.at[pl.ds(s * _ENT_SLICE + bb * _ZROWS, _ZROWS), :])

        @pl.when(c == 0)
        def _():
            pltpu.sync_copy(zbuf1, acc_c.at[pl.ds(s * 640, 640)])
        plsc.subcore_barrier()

        run_phase(kg=True)
        plsc.subcore_barrier()

        sl = pl.ds(s * _ENT_SLICE, _ENT_SLICE)

        @pl.when(c == 0)
        def _():
            pltpu.sync_copy(acc.at[sl, :], s0.at[sl, :])

        @pl.when(c == 1)
        def _():
            pltpu.sync_copy(acc.at[sl, :], s1.at[sl, :])

        @pl.when(c == 0)
        def _():
            pltpu.sync_copy(acc_c.at[pl.ds(s * 640, 640)],
                            cnt_out.at[pl.ds(s * 640, 640)])
        plsc.subcore_barrier()

        for b in range(_USR_SLICE // _ZROWS):
            pltpu.sync_copy(
                zbuf, acc.at[pl.ds(s * _USR_SLICE + b * _ZROWS, _ZROWS), :])
        plsc.subcore_barrier()

        run_phase(kg=False)
        plsc.subcore_barrier()

        for b in range(_USR_SLICE // _ZROWS):
            slb = pl.ds(s * _USR_SLICE + b * _ZROWS, _ZROWS)

            @pl.when(c == 0)
            def _():
                pltpu.sync_copy(acc.at[slb, :], u0.at[slb, :])

            @pl.when(c == 1)
            def _():
                pltpu.sync_copy(acc.at[slb, :], u1.at[slb, :])

    return pl.kernel(body, out_type=tuple(out_type), mesh=mesh,
                     scratch_types=scratch,
                     compiler_params=pltpu.CompilerParams(
                         use_tc_tiling_on_sc=False))


_sc_cache = {}


def _sc_kernel():
    if "k" not in _sc_cache:
        _sc_cache["k"] = _make_sc_kernel()
    return _sc_cache["k"]


_BE = 1000


def _entity_body(s0, s1, cnt, res, enew, rout):
    sfull = jnp.concatenate([s0[...], s1[...]], axis=1)
    cv = jnp.maximum(cnt[...], 1.0)
    agg = sfull / cv
    nrm = jnp.sqrt(jnp.sum(agg * agg, axis=1, keepdims=True))
    e = agg / jnp.maximum(nrm, 1e-12)
    enew[...] = e
    rout[...] = res[...] + e


def _tc_entity(s0, s1, cnt, res_in):
    n = res_in.shape[0]
    grid = (n // _BE,)
    return pl.pallas_call(
        _entity_body,
        grid=grid,
        in_specs=[
            pl.BlockSpec((_BE, HD), lambda i: (i, 0)),
            pl.BlockSpec((_BE, HD), lambda i: (i, 0)),
            pl.BlockSpec((_BE, 1), lambda i: (i, 0)),
            pl.BlockSpec((_BE, D), lambda i: (i, 0)),
        ],
        out_specs=[
            pl.BlockSpec((_BE, D), lambda i: (i, 0)),
            pl.BlockSpec((_BE, D), lambda i: (i, 0)),
        ],
        out_shape=[
            jax.ShapeDtypeStruct((n, D), jnp.float32),
            jax.ShapeDtypeStruct((n, D), jnp.float32),
        ],
    )(s0, s1, cnt, res_in)


def _user_body(u0, u1, uemb, latr, dwr, res, unew, rout):
    ua = jnp.concatenate([u0[...], u1[...]], axis=1)
    logits = lax.dot_general(
        uemb[...], latr[...], (((1,), (1,)), ((), ())),
        preferred_element_type=jnp.float32, precision=lax.Precision.HIGHEST)
    m = jnp.max(logits, axis=1, keepdims=True)
    p = jnp.exp(logits - m)
    p = p / jnp.sum(p, axis=1, keepdims=True)
    factor = lax.dot_general(
        p, dwr[...], (((1,), (0,)), ((), ())),
        preferred_element_type=jnp.float32, precision=lax.Precision.HIGHEST)
    out = factor * ua + ua
    nrm = jnp.sqrt(jnp.sum(out * out, axis=1, keepdims=True))
    u = out / jnp.maximum(nrm, 1e-12)
    unew[...] = u
    rout[...] = res[...] + u


def _tc_user(u0, u1, uemb, latent, dw, res_in):
    n = uemb.shape[0]
    grid = (n // _BE,)
    nf = latent.shape[0]
    return pl.pallas_call(
        _user_body,
        grid=grid,
        in_specs=[
            pl.BlockSpec((_BE, HD), lambda i: (i, 0)),
            pl.BlockSpec((_BE, HD), lambda i: (i, 0)),
            pl.BlockSpec((_BE, D), lambda i: (i, 0)),
            pl.BlockSpec((nf, D), lambda i: (0, 0)),
            pl.BlockSpec((nf, D), lambda i: (0, 0)),
            pl.BlockSpec((_BE, D), lambda i: (i, 0)),
        ],
        out_specs=[
            pl.BlockSpec((_BE, D), lambda i: (i, 0)),
            pl.BlockSpec((_BE, D), lambda i: (i, 0)),
        ],
        out_shape=[
            jax.ShapeDtypeStruct((n, D), jnp.float32),
            jax.ShapeDtypeStruct((n, D), jnp.float32),
        ],
    )(u0, u1, uemb, latent, dw, res_in)


def _prep_body(att_ref, rel_ref, dw_ref, cor_ref):
    att = att_ref[...]
    m = jnp.max(att, axis=1, keepdims=True)
    p = jnp.exp(att - m)
    p = p / jnp.sum(p, axis=1, keepdims=True)
    dw_ref[...] = lax.dot_general(
        p, rel_ref[...], (((1,), (0,)), ((), ())),
        preferred_element_type=jnp.float32, precision=lax.Precision.HIGHEST)
    nrm = jnp.sqrt(jnp.sum(att * att, axis=1, keepdims=True))
    nt = att / jnp.maximum(nrm, 1e-12)
    sim = lax.dot_general(
        nt, nt, (((1,), (1,)), ((), ())),
        preferred_element_type=jnp.float32, precision=lax.Precision.HIGHEST)
    sc = jnp.exp(sim / 0.2)
    rows = jnp.sum(sc, axis=1)
    nf = att.shape[0]
    ii = lax.broadcasted_iota(jnp.int32, (nf, nf), 0)
    jj = lax.broadcasted_iota(jnp.int32, (nf, nf), 1)
    diag = jnp.sum(jnp.where(ii == jj, sc, 0.0), axis=1)
    cor_ref[...] = (-jnp.sum(jnp.log(diag) - jnp.log(rows))).reshape(1, 1)


def _tc_prep(att, rel):
    nf, nr = att.shape
    return pl.pallas_call(
        _prep_body,
        out_shape=[
            jax.ShapeDtypeStruct((nf, D), jnp.float32),
            jax.ShapeDtypeStruct((1, 1), jnp.float32),
        ],
    )(att, rel)


def kernel(user_emb, entity_emb, latent_emb, relation_emb, disen_weight_att,
           interact_values, edge_index, edge_type, interact_row, interact_col):
    headr = edge_index[0].reshape(KG_SUPERS, SUB, C)
    tailr = edge_index[1].reshape(KG_SUPERS, SUB, C)
    typer = edge_type.reshape(KG_SUPERS, SUB, C)
    colr = interact_col.reshape(U_SUPERS, SUB, C)
    urowr = interact_row.reshape(U_SUPERS, SUB, C)
    uvalr = interact_values.reshape(U_SUPERS, SUB, C)
    relL = relation_emb[:, :HD]
    relR = relation_emb[:, HD:]

    entL = entity_emb[:, :HD]
    entR = entity_emb[:, HD:]
    s0, s1, u0, u1, cnt = _sc_kernel()(
        entL, entR, relL, relR, tailr, typer, headr, colr, urowr, uvalr)
    cnt2d = cnt.reshape(N_ENT_P, 1)
    e1, eres1 = _tc_entity(s0, s1, cnt2d, entity_emb)
    dw, cor = _tc_prep(disen_weight_att, relation_emb)
    u1n, ures1 = _tc_user(u0, u1, user_emb, latent_emb, dw, user_emb)

    s0b, s1b, u0b, u1b, _cnt2 = _sc_kernel()(
        e1[:, :HD], e1[:, HD:], relL, relR, tailr, typer, headr, colr,
        urowr, uvalr)
    e2, eres2 = _tc_entity(s0b, s1b, cnt2d, eres1)
    u2n, ures2 = _tc_user(u0b, u1b, u1n, latent_emb, dw, ures1)

    return (eres2, ures2, cor.reshape(()))


# speedup vs baseline: 4.7172x; 1.3554x over previous
import functools

import jax
import jax.numpy as jnp
from jax import lax
from jax.experimental import pallas as pl
from jax.experimental.pallas import tpu as pltpu
from jax.experimental.pallas import tpu_sc as plsc

N_ENT = 10000
N_USR = 20000
N_ENT_P = 10240
N_USR_P = 20480
D = 128
HD = 64
N_REL = 16
N_EDGE = 320000
NNZ = 500000
C = 80
SUB = 10

KG_SUPERS = N_EDGE // (C * SUB)
KG_SUPERS_PER_TILE = KG_SUPERS // 16
U_SUPERS = NNZ // (C * SUB)
U_SUPERS_PER_TILE = -(-U_SUPERS // 16)

_ENT_SLICE = N_ENT_P // 16
_USR_SLICE = N_USR_P // 16
_ZROWS = 160


def _fill_zero_2d(ref, rows):
    def body(i, _):
        for j in range(HD // 16):
            ref[i, pl.ds(j * 16, 16)] = jnp.zeros((16,), jnp.float32)
        return 0
    lax.fori_loop(0, rows, body, 0)


def _fill_const_1d(ref, n16, val):
    def body(i, _):
        ref[pl.ds(i * 16, 16)] = jnp.full((16,), val, jnp.float32)
        return 0
    lax.fori_loop(0, n16, body, 0)


def _make_sc_kernel(with_cnt: bool):
    mesh = plsc.VectorSubcoreMesh(core_axis_name="c", subcore_axis_name="s",
                                  num_cores=2, num_subcores=16)

    out_type = [
        jax.ShapeDtypeStruct((N_ENT_P, HD), jnp.float32),
        jax.ShapeDtypeStruct((N_ENT_P, HD), jnp.float32),
        jax.ShapeDtypeStruct((N_USR_P, HD), jnp.float32),
        jax.ShapeDtypeStruct((N_USR_P, HD), jnp.float32),
        jax.ShapeDtypeStruct((N_ENT_P,), jnp.float32),
    ]

    scratch = dict(
        acc=pltpu.VMEM_SHARED((N_USR_P, HD), jnp.float32),
        t_idx=pltpu.VMEM((SUB, C), jnp.int32),
        h_idx=pltpu.VMEM((SUB, C), jnp.int32),
        v_buf=pltpu.VMEM((SUB, C), jnp.float32),
        rows0=pltpu.VMEM((C, HD), jnp.float32),
        rows1=pltpu.VMEM((C, HD), jnp.float32),
        zbuf=pltpu.VMEM((_ZROWS, HD), jnp.float32),
        ones=pltpu.VMEM((C,), jnp.float32),
        sem_i=pltpu.SemaphoreType.DMA,
        sem_g=pltpu.SemaphoreType.DMA,
        sem_s0=pltpu.SemaphoreType.DMA,
        sem_s1=pltpu.SemaphoreType.DMA,
        acc_c=pltpu.VMEM_SHARED((N_ENT_P,), jnp.float32),
        zbuf1=pltpu.VMEM((640,), jnp.float32),
    )

    def body(tabL, tabR, entL, entR, kgidxr, headr, colr, urowr, uvalr,
             *refs, acc, t_idx, h_idx, v_buf, rows0, rows1,
             zbuf, ones, sem_i, sem_g, sem_s0, sem_s1, acc_c, zbuf1):
        s0, s1, u0, u1, cnt_out = refs

        c = lax.axis_index("c")
        s = lax.axis_index("s")

        _fill_zero_2d(zbuf, _ZROWS)
        _fill_const_1d(ones, C // 16, 1.0)
        _fill_const_1d(zbuf1, 40, 0.0)

        def gather_start_from(srcL, srcR, idx_row, buf):
            @pl.when(c == 0)
            def _():
                pltpu.async_copy(srcL.at[idx_row], buf, sem_g)

            @pl.when(c == 1)
            def _():
                pltpu.async_copy(srcR.at[idx_row], buf, sem_g)

        def run_phase(kg):
            nsup = KG_SUPERS_PER_TILE if kg else U_SUPERS_PER_TILE
            iarr = kgidxr if kg else colr
            harr = headr if kg else urowr
            bufs = (rows0, rows1)
            ssem = (sem_s0, sem_s1)

            def gather_start(j, buf):
                if kg:
                    gather_start_from(tabL, tabR, t_idx.at[j], buf)
                else:
                    gather_start_from(entL, entR, t_idx.at[j], buf)

            def gather_wait(buf):
                pltpu.make_async_copy(entL.at[t_idx.at[0]], buf, sem_g).wait()

            def scatter_wait(j):
                pltpu.make_async_copy(bufs[j % 2], acc.at[h_idx.at[0]],
                                      ssem[j % 2]).wait()

            def scale_chunk(j, buf):
                def scale_body(g, _):
                    vv = v_buf[j, pl.ds(g * 16, 16)]
                    svs = [vv[r2] for r2 in range(16)]
                    for r2 in range(16):
                        row = g * 16 + r2
                        es = [buf[row, pl.ds(jj * 16, 16)]
                              for jj in range(HD // 16)]
                        for jj in range(HD // 16):
                            buf[row, pl.ds(jj * 16, 16)] = es[jj] * svs[r2]
                    return 0
                lax.fori_loop(0, C // 16, scale_body, 0)

            def run_super(m):
                sid = (s * KG_SUPERS_PER_TILE + m) if kg else (s + 16 * m)
                pltpu.async_copy(iarr.at[sid], t_idx, sem_i)
                pltpu.async_copy(harr.at[sid], h_idx, sem_i)
                if not kg:
                    pltpu.async_copy(uvalr.at[sid], v_buf, sem_i)
                pltpu.make_async_copy(iarr.at[0], t_idx, sem_i).wait()
                pltpu.make_async_copy(harr.at[0], h_idx, sem_i).wait()
                if not kg:
                    pltpu.make_async_copy(uvalr.at[0], v_buf, sem_i).wait()

                gather_start(0, bufs[0])
                for j in range(SUB):
                    b = j % 2
                    gather_wait(bufs[b])
                    if j > 0:
                        scatter_wait(j - 1)
                    if j < SUB - 1:
                        gather_start(j + 1, bufs[1 - b])
                    if not kg:
                        scale_chunk(j, bufs[b])
                    pltpu.async_copy(bufs[b], acc.at[h_idx.at[j]], ssem[b],
                                     add=True)
                    if kg and with_cnt:
                        @pl.when(c == 0)
                        def _():
                            pltpu.sync_copy(ones, acc_c.at[h_idx.at[j]],
                                            add=True)
                scatter_wait(SUB - 1)

            def sup_body(m, _):
                if kg:
                    run_super(m)
                else:
                    @pl.when(s + 16 * m < U_SUPERS)
                    def _():
                        run_super(m)
                return 0
            lax.fori_loop(0, nsup, sup_body, 0)

        for bb in range(_ENT_SLICE // _ZROWS):
            pltpu.sync_copy(
                zbuf, acc.at[pl.ds(s * _ENT_SLICE + bb * _ZROWS, _ZROWS), :])

        if with_cnt:
            @pl.when(c == 0)
            def _():
                pltpu.sync_copy(zbuf1, acc_c.at[pl.ds(s * 640, 640)])
        plsc.subcore_barrier()

        run_phase(kg=True)
        plsc.subcore_barrier()

        sl = pl.ds(s * _ENT_SLICE, _ENT_SLICE)

        @pl.when(c == 0)
        def _():
            pltpu.sync_copy(acc.at[sl, :], s0.at[sl, :])

        @pl.when(c == 1)
        def _():
            pltpu.sync_copy(acc.at[sl, :], s1.at[sl, :])

        if with_cnt:
            @pl.when(c == 0)
            def _():
                pltpu.sync_copy(acc_c.at[pl.ds(s * 640, 640)],
                                cnt_out.at[pl.ds(s * 640, 640)])
        plsc.subcore_barrier()

        for b in range(_USR_SLICE // _ZROWS):
            pltpu.sync_copy(
                zbuf, acc.at[pl.ds(s * _USR_SLICE + b * _ZROWS, _ZROWS), :])
        plsc.subcore_barrier()

        run_phase(kg=False)
        plsc.subcore_barrier()

        for b in range(_USR_SLICE // _ZROWS):
            slb = pl.ds(s * _USR_SLICE + b * _ZROWS, _ZROWS)

            @pl.when(c == 0)
            def _():
                pltpu.sync_copy(acc.at[slb, :], u0.at[slb, :])

            @pl.when(c == 1)
            def _():
                pltpu.sync_copy(acc.at[slb, :], u1.at[slb, :])

    return pl.kernel(body, out_type=tuple(out_type), mesh=mesh,
                     scratch_types=scratch,
                     compiler_params=pltpu.CompilerParams(
                         use_tc_tiling_on_sc=False))


_sc_cache = {}


def _sc_kernel(with_cnt: bool):
    if with_cnt not in _sc_cache:
        _sc_cache[with_cnt] = _make_sc_kernel(with_cnt)
    return _sc_cache[with_cnt]


_BE = 1000
_BT = 1000


def _tables_body(ent, rel, outL, outR):
    e = ent[...]
    t = pl.program_id(1)
    tids = lax.broadcasted_iota(jnp.int32, (N_REL, 1), 0)
    r = jnp.sum(jnp.where(tids == t, rel[...], 0.0), axis=0,
                keepdims=True)
    outL[...] = e[:, :HD] * r[:, :HD]
    outR[...] = e[:, HD:] * r[:, HD:]


def _tc_tables(ent, rel):
    ni = N_ENT // _BT
    return pl.pallas_call(
        _tables_body,
        grid=(ni, N_REL),
        in_specs=[
            pl.BlockSpec((_BT, D), lambda i, t: (i, 0)),
            pl.BlockSpec((N_REL, D), lambda i, t: (0, 0)),
        ],
        out_specs=[
            pl.BlockSpec((_BT, HD), lambda i, t: (t * (N_ENT // _BT) + i, 0)),
            pl.BlockSpec((_BT, HD), lambda i, t: (t * (N_ENT // _BT) + i, 0)),
        ],
        out_shape=[
            jax.ShapeDtypeStruct((N_REL * N_ENT, HD), jnp.float32),
            jax.ShapeDtypeStruct((N_REL * N_ENT, HD), jnp.float32),
        ],
    )(ent, rel)


def _entity_body(s0, s1, cnt, res, enew, rout):
    sfull = jnp.concatenate([s0[...], s1[...]], axis=1)
    cv = jnp.maximum(cnt[...], 1.0)
    agg = sfull / cv
    nrm = jnp.sqrt(jnp.sum(agg * agg, axis=1, keepdims=True))
    e = agg / jnp.maximum(nrm, 1e-12)
    enew[...] = e
    rout[...] = res[...] + e


def _tc_entity(s0, s1, cnt, res_in):
    n = res_in.shape[0]
    grid = (n // _BE,)
    return pl.pallas_call(
        _entity_body,
        grid=grid,
        in_specs=[
            pl.BlockSpec((_BE, HD), lambda i: (i, 0)),
            pl.BlockSpec((_BE, HD), lambda i: (i, 0)),
            pl.BlockSpec((_BE, 1), lambda i: (i, 0)),
            pl.BlockSpec((_BE, D), lambda i: (i, 0)),
        ],
        out_specs=[
            pl.BlockSpec((_BE, D), lambda i: (i, 0)),
            pl.BlockSpec((_BE, D), lambda i: (i, 0)),
        ],
        out_shape=[
            jax.ShapeDtypeStruct((n, D), jnp.float32),
            jax.ShapeDtypeStruct((n, D), jnp.float32),
        ],
    )(s0, s1, cnt, res_in)


def _user_body(u0, u1, uemb, latr, dwr, res, unew, rout):
    ua = jnp.concatenate([u0[...], u1[...]], axis=1)
    logits = lax.dot_general(
        uemb[...], latr[...], (((1,), (1,)), ((), ())),
        preferred_element_type=jnp.float32, precision=lax.Precision.HIGHEST)
    m = jnp.max(logits, axis=1, keepdims=True)
    p = jnp.exp(logits - m)
    p = p / jnp.sum(p, axis=1, keepdims=True)
    factor = lax.dot_general(
        p, dwr[...], (((1,), (0,)), ((), ())),
        preferred_element_type=jnp.float32, precision=lax.Precision.HIGHEST)
    out = factor * ua + ua
    nrm = jnp.sqrt(jnp.sum(out * out, axis=1, keepdims=True))
    u = out / jnp.maximum(nrm, 1e-12)
    unew[...] = u
    rout[...] = res[...] + u


def _tc_user(u0, u1, uemb, latent, dw, res_in):
    n = uemb.shape[0]
    grid = (n // _BE,)
    nf = latent.shape[0]
    return pl.pallas_call(
        _user_body,
        grid=grid,
        in_specs=[
            pl.BlockSpec((_BE, HD), lambda i: (i, 0)),
            pl.BlockSpec((_BE, HD), lambda i: (i, 0)),
            pl.BlockSpec((_BE, D), lambda i: (i, 0)),
            pl.BlockSpec((nf, D), lambda i: (0, 0)),
            pl.BlockSpec((nf, D), lambda i: (0, 0)),
            pl.BlockSpec((_BE, D), lambda i: (i, 0)),
        ],
        out_specs=[
            pl.BlockSpec((_BE, D), lambda i: (i, 0)),
            pl.BlockSpec((_BE, D), lambda i: (i, 0)),
        ],
        out_shape=[
            jax.ShapeDtypeStruct((n, D), jnp.float32),
            jax.ShapeDtypeStruct((n, D), jnp.float32),
        ],
    )(u0, u1, uemb, latent, dw, res_in)


def _prep_body(att_ref, rel_ref, dw_ref, cor_ref):
    att = att_ref[...]
    m = jnp.max(att, axis=1, keepdims=True)
    p = jnp.exp(att - m)
    p = p / jnp.sum(p, axis=1, keepdims=True)
    dw_ref[...] = lax.dot_general(
        p, rel_ref[...], (((1,), (0,)), ((), ())),
        preferred_element_type=jnp.float32, precision=lax.Precision.HIGHEST)
    nrm = jnp.sqrt(jnp.sum(att * att, axis=1, keepdims=True))
    nt = att / jnp.maximum(nrm, 1e-12)
    sim = lax.dot_general(
        nt, nt, (((1,), (1,)), ((), ())),
        preferred_element_type=jnp.float32, precision=lax.Precision.HIGHEST)
    sc = jnp.exp(sim / 0.2)
    rows = jnp.sum(sc, axis=1)
    nf = att.shape[0]
    ii = lax.broadcasted_iota(jnp.int32, (nf, nf), 0)
    jj = lax.broadcasted_iota(jnp.int32, (nf, nf), 1)
    diag = jnp.sum(jnp.where(ii == jj, sc, 0.0), axis=1)
    cor_ref[...] = (-jnp.sum(jnp.log(diag) - jnp.log(rows))).reshape(1, 1)


def _tc_prep(att, rel):
    nf, nr = att.shape
    return pl.pallas_call(
        _prep_body,
        out_shape=[
            jax.ShapeDtypeStruct((nf, D), jnp.float32),
            jax.ShapeDtypeStruct((1, 1), jnp.float32),
        ],
    )(att, rel)


def kernel(user_emb, entity_emb, latent_emb, relation_emb, disen_weight_att,
           interact_values, edge_index, edge_type, interact_row, interact_col):
    headr = edge_index[0].reshape(KG_SUPERS, SUB, C)
    kgidxr = (edge_type * N_ENT + edge_index[1]).reshape(KG_SUPERS, SUB, C)
    colr = interact_col.reshape(U_SUPERS, SUB, C)
    urowr = interact_row.reshape(U_SUPERS, SUB, C)
    uvalr = interact_values.reshape(U_SUPERS, SUB, C)

    tabL, tabR = _tc_tables(entity_emb, relation_emb)
    s0, s1, u0, u1, cnt = _sc_kernel(True)(
        tabL, tabR, entity_emb[:, :HD], entity_emb[:, HD:],
        kgidxr, headr, colr, urowr, uvalr)
    cnt2d = cnt.reshape(N_ENT_P, 1)
    e1, eres1 = _tc_entity(s0, s1, cnt2d, entity_emb)
    dw, cor = _tc_prep(disen_weight_att, relation_emb)
    u1n, ures1 = _tc_user(u0, u1, user_emb, latent_emb, dw, user_emb)

    tabL2, tabR2 = _tc_tables(e1, relation_emb)
    s0b, s1b, u0b, u1b, _cnt2 = _sc_kernel(False)(
        tabL2, tabR2, e1[:, :HD], e1[:, HD:],
        kgidxr, headr, colr, urowr, uvalr)
    e2, eres2 = _tc_entity(s0b, s1b, cnt2d, eres1)
    u2n, ures2 = _tc_user(u0b, u1b, u1n, latent_emb, dw, ures1)

    return (eres2, ures2, cor.reshape(()))


# speedup vs baseline: 5.7490x; 1.2187x over previous
import functools

import jax
import jax.numpy as jnp
from jax import lax
from jax.experimental import pallas as pl
from jax.experimental.pallas import tpu as pltpu
from jax.experimental.pallas import tpu_sc as plsc

N_ENT = 10000
N_USR = 20000
N_ENT_P = 10240
N_USR_P = 20480
D = 128
HD = 64
N_REL = 16
N_EDGE = 320000
NNZ = 500000
C = 80
SUB = 10

KG_SUPERS = N_EDGE // (C * SUB)
KG_SUPERS_PER_TILE = KG_SUPERS // 16
U_SUPERS = NNZ // (C * SUB)
U_SUPERS_PER_TILE = -(-U_SUPERS // 16)

_ENT_SLICE = N_ENT_P // 16
_USR_SLICE = N_USR_P // 16
_ZROWS = 160


def _fill_zero_2d(ref, rows):
    def body(i, _):
        for j in range(HD // 16):
            ref[i, pl.ds(j * 16, 16)] = jnp.zeros((16,), jnp.float32)
        return 0
    lax.fori_loop(0, rows, body, 0)


def _fill_const_1d(ref, n16, val):
    def body(i, _):
        ref[pl.ds(i * 16, 16)] = jnp.full((16,), val, jnp.float32)
        return 0
    lax.fori_loop(0, n16, body, 0)


def _make_sc_kernel(with_cnt: bool):
    mesh = plsc.VectorSubcoreMesh(core_axis_name="c", subcore_axis_name="s",
                                  num_cores=2, num_subcores=16)

    out_type = [
        jax.ShapeDtypeStruct((N_ENT_P, HD), jnp.float32),
        jax.ShapeDtypeStruct((N_ENT_P, HD), jnp.float32),
        jax.ShapeDtypeStruct((N_USR_P, HD), jnp.float32),
        jax.ShapeDtypeStruct((N_USR_P, HD), jnp.float32),
        jax.ShapeDtypeStruct((N_ENT_P,), jnp.float32),
    ]

    scratch = dict(
        acc=pltpu.VMEM_SHARED((N_USR_P, HD), jnp.float32),
        t_idx=pltpu.VMEM((SUB, C), jnp.int32),
        h_idx=pltpu.VMEM((SUB, C), jnp.int32),
        v_buf=pltpu.VMEM((SUB, C), jnp.float32),
        rows0=pltpu.VMEM((C, HD), jnp.float32),
        rows1=pltpu.VMEM((C, HD), jnp.float32),
        rows2=pltpu.VMEM((C, HD), jnp.float32),
        zbuf=pltpu.VMEM((_ZROWS, HD), jnp.float32),
        ones=pltpu.VMEM((C,), jnp.float32),
        sem_i=pltpu.SemaphoreType.DMA,
        sem_g0=pltpu.SemaphoreType.DMA,
        sem_g1=pltpu.SemaphoreType.DMA,
        sem_g2=pltpu.SemaphoreType.DMA,
        sem_s0=pltpu.SemaphoreType.DMA,
        sem_s1=pltpu.SemaphoreType.DMA,
        sem_s2=pltpu.SemaphoreType.DMA,
        acc_c=pltpu.VMEM_SHARED((N_ENT_P,), jnp.float32),
        zbuf1=pltpu.VMEM((640,), jnp.float32),
    )

    def body(tabL, tabR, entL, entR, kgidxr, headr, colr, urowr, uvalr,
             *refs, acc, t_idx, h_idx, v_buf, rows0, rows1, rows2,
             zbuf, ones, sem_i, sem_g0, sem_g1, sem_g2, sem_s0, sem_s1,
             sem_s2, acc_c, zbuf1):
        s0, s1, u0, u1, cnt_out = refs

        c = lax.axis_index("c")
        s = lax.axis_index("s")

        _fill_zero_2d(zbuf, _ZROWS)
        _fill_const_1d(ones, C // 16, 1.0)
        _fill_const_1d(zbuf1, 40, 0.0)

        def gather_start_from(srcL, srcR, idx_row, buf, sem):
            @pl.when(c == 0)
            def _():
                pltpu.async_copy(srcL.at[idx_row], buf, sem)

            @pl.when(c == 1)
            def _():
                pltpu.async_copy(srcR.at[idx_row], buf, sem)

        def run_phase(kg):
            nsup = KG_SUPERS_PER_TILE if kg else U_SUPERS_PER_TILE
            iarr = kgidxr if kg else colr
            harr = headr if kg else urowr
            bufs = (rows0, rows1, rows2)
            gsem = (sem_g0, sem_g1, sem_g2)
            ssem = (sem_s0, sem_s1, sem_s2)

            def gather_start(j):
                if kg:
                    gather_start_from(tabL, tabR, t_idx.at[j], bufs[j % 3],
                                      gsem[j % 3])
                else:
                    gather_start_from(entL, entR, t_idx.at[j], bufs[j % 3],
                                      gsem[j % 3])

            def gather_wait(j):
                pltpu.make_async_copy(entL.at[t_idx.at[0]], bufs[j % 3],
                                      gsem[j % 3]).wait()

            def scatter_wait(j):
                pltpu.make_async_copy(bufs[j % 3], acc.at[h_idx.at[0]],
                                      ssem[j % 3]).wait()

            def scale_chunk(j, buf):
                def scale_body(g, _):
                    vv = v_buf[j, pl.ds(g * 16, 16)]
                    svs = [vv[r2] for r2 in range(16)]
                    for r2 in range(16):
                        row = g * 16 + r2
                        es = [buf[row, pl.ds(jj * 16, 16)]
                              for jj in range(HD // 16)]
                        for jj in range(HD // 16):
                            buf[row, pl.ds(jj * 16, 16)] = es[jj] * svs[r2]
                    return 0
                lax.fori_loop(0, C // 16, scale_body, 0)

            def run_super(m):
                sid = (s * KG_SUPERS_PER_TILE + m) if kg else (s + 16 * m)
                pltpu.async_copy(iarr.at[sid], t_idx, sem_i)
                pltpu.async_copy(harr.at[sid], h_idx, sem_i)
                if not kg:
                    pltpu.async_copy(uvalr.at[sid], v_buf, sem_i)
                pltpu.make_async_copy(iarr.at[0], t_idx, sem_i).wait()
                pltpu.make_async_copy(harr.at[0], h_idx, sem_i).wait()
                if not kg:
                    pltpu.make_async_copy(uvalr.at[0], v_buf, sem_i).wait()

                gather_start(0)
                gather_start(1)
                for j in range(SUB):
                    b = j % 3
                    gather_wait(j)
                    if j + 2 < SUB:
                        if j > 0:
                            scatter_wait(j - 1)
                        gather_start(j + 2)
                    if not kg:
                        scale_chunk(j, bufs[b])
                    pltpu.async_copy(bufs[b], acc.at[h_idx.at[j]], ssem[b],
                                     add=True)
                    if kg and with_cnt:
                        @pl.when(c == 0)
                        def _():
                            pltpu.sync_copy(ones, acc_c.at[h_idx.at[j]],
                                            add=True)
                scatter_wait(SUB - 2)
                scatter_wait(SUB - 1)

            def sup_body(m, _):
                if kg:
                    run_super(m)
                else:
                    @pl.when(s + 16 * m < U_SUPERS)
                    def _():
                        run_super(m)
                return 0
            lax.fori_loop(0, nsup, sup_body, 0)

        for bb in range(_ENT_SLICE // _ZROWS):
            pltpu.sync_copy(
                zbuf, acc.at[pl.ds(s * _ENT_SLICE + bb * _ZROWS, _ZROWS), :])

        if with_cnt:
            @pl.when(c == 0)
            def _():
                pltpu.sync_copy(zbuf1, acc_c.at[pl.ds(s * 640, 640)])
        plsc.subcore_barrier()

        run_phase(kg=True)
        plsc.subcore_barrier()

        sl = pl.ds(s * _ENT_SLICE, _ENT_SLICE)

        @pl.when(c == 0)
        def _():
            pltpu.sync_copy(acc.at[sl, :], s0.at[sl, :])

        @pl.when(c == 1)
        def _():
            pltpu.sync_copy(acc.at[sl, :], s1.at[sl, :])

        if with_cnt:
            @pl.when(c == 0)
            def _():
                pltpu.sync_copy(acc_c.at[pl.ds(s * 640, 640)],
                                cnt_out.at[pl.ds(s * 640, 640)])
        plsc.subcore_barrier()

        for b in range(_USR_SLICE // _ZROWS):
            pltpu.sync_copy(
                zbuf, acc.at[pl.ds(s * _USR_SLICE + b * _ZROWS, _ZROWS), :])
        plsc.subcore_barrier()

        run_phase(kg=False)
        plsc.subcore_barrier()

        for b in range(_USR_SLICE // _ZROWS):
            slb = pl.ds(s * _USR_SLICE + b * _ZROWS, _ZROWS)

            @pl.when(c == 0)
            def _():
                pltpu.sync_copy(acc.at[slb, :], u0.at[slb, :])

            @pl.when(c == 1)
            def _():
                pltpu.sync_copy(acc.at[slb, :], u1.at[slb, :])

    return pl.kernel(body, out_type=tuple(out_type), mesh=mesh,
                     scratch_types=scratch,
                     compiler_params=pltpu.CompilerParams(
                         use_tc_tiling_on_sc=False))


_sc_cache = {}


def _sc_kernel(with_cnt: bool):
    if with_cnt not in _sc_cache:
        _sc_cache[with_cnt] = _make_sc_kernel(with_cnt)
    return _sc_cache[with_cnt]


_BE = 1000
_BT = 1000


def _tables_body(ent, rel, outL, outR):
    e = ent[...]
    t = pl.program_id(1)
    tids = lax.broadcasted_iota(jnp.int32, (N_REL, 1), 0)
    r = jnp.sum(jnp.where(tids == t, rel[...], 0.0), axis=0,
                keepdims=True)
    outL[...] = e[:, :HD] * r[:, :HD]
    outR[...] = e[:, HD:] * r[:, HD:]


def _tc_tables(ent, rel):
    ni = N_ENT // _BT
    return pl.pallas_call(
        _tables_body,
        grid=(ni, N_REL),
        in_specs=[
            pl.BlockSpec((_BT, D), lambda i, t: (i, 0)),
            pl.BlockSpec((N_REL, D), lambda i, t: (0, 0)),
        ],
        out_specs=[
            pl.BlockSpec((_BT, HD), lambda i, t: (t * (N_ENT // _BT) + i, 0)),
            pl.BlockSpec((_BT, HD), lambda i, t: (t * (N_ENT // _BT) + i, 0)),
        ],
        out_shape=[
            jax.ShapeDtypeStruct((N_REL * N_ENT, HD), jnp.float32),
            jax.ShapeDtypeStruct((N_REL * N_ENT, HD), jnp.float32),
        ],
    )(ent, rel)


def _entity_body(s0, s1, cnt, res, enew, rout):
    sfull = jnp.concatenate([s0[...], s1[...]], axis=1)
    cv = jnp.maximum(cnt[...], 1.0)
    agg = sfull / cv
    nrm = jnp.sqrt(jnp.sum(agg * agg, axis=1, keepdims=True))
    e = agg / jnp.maximum(nrm, 1e-12)
    enew[...] = e
    rout[...] = res[...] + e


def _tc_entity(s0, s1, cnt, res_in):
    n = res_in.shape[0]
    grid = (n // _BE,)
    return pl.pallas_call(
        _entity_body,
        grid=grid,
        in_specs=[
            pl.BlockSpec((_BE, HD), lambda i: (i, 0)),
            pl.BlockSpec((_BE, HD), lambda i: (i, 0)),
            pl.BlockSpec((_BE, 1), lambda i: (i, 0)),
            pl.BlockSpec((_BE, D), lambda i: (i, 0)),
        ],
        out_specs=[
            pl.BlockSpec((_BE, D), lambda i: (i, 0)),
            pl.BlockSpec((_BE, D), lambda i: (i, 0)),
        ],
        out_shape=[
            jax.ShapeDtypeStruct((n, D), jnp.float32),
            jax.ShapeDtypeStruct((n, D), jnp.float32),
        ],
    )(s0, s1, cnt, res_in)


def _user_body(u0, u1, uemb, latr, dwr, res, unew, rout):
    ua = jnp.concatenate([u0[...], u1[...]], axis=1)
    logits = lax.dot_general(
        uemb[...], latr[...], (((1,), (1,)), ((), ())),
        preferred_element_type=jnp.float32, precision=lax.Precision.HIGHEST)
    m = jnp.max(logits, axis=1, keepdims=True)
    p = jnp.exp(logits - m)
    p = p / jnp.sum(p, axis=1, keepdims=True)
    factor = lax.dot_general(
        p, dwr[...], (((1,), (0,)), ((), ())),
        preferred_element_type=jnp.float32, precision=lax.Precision.HIGHEST)
    out = factor * ua + ua
    nrm = jnp.sqrt(jnp.sum(out * out, axis=1, keepdims=True))
    u = out / jnp.maximum(nrm, 1e-12)
    unew[...] = u
    rout[...] = res[...] + u


def _tc_user(u0, u1, uemb, latent, dw, res_in):
    n = uemb.shape[0]
    grid = (n // _BE,)
    nf = latent.shape[0]
    return pl.pallas_call(
        _user_body,
        grid=grid,
        in_specs=[
            pl.BlockSpec((_BE, HD), lambda i: (i, 0)),
            pl.BlockSpec((_BE, HD), lambda i: (i, 0)),
            pl.BlockSpec((_BE, D), lambda i: (i, 0)),
            pl.BlockSpec((nf, D), lambda i: (0, 0)),
            pl.BlockSpec((nf, D), lambda i: (0, 0)),
            pl.BlockSpec((_BE, D), lambda i: (i, 0)),
        ],
        out_specs=[
            pl.BlockSpec((_BE, D), lambda i: (i, 0)),
            pl.BlockSpec((_BE, D), lambda i: (i, 0)),
        ],
        out_shape=[
            jax.ShapeDtypeStruct((n, D), jnp.float32),
            jax.ShapeDtypeStruct((n, D), jnp.float32),
        ],
    )(u0, u1, uemb, latent, dw, res_in)


def _prep_body(att_ref, rel_ref, dw_ref, cor_ref):
    att = att_ref[...]
    m = jnp.max(att, axis=1, keepdims=True)
    p = jnp.exp(att - m)
    p = p / jnp.sum(p, axis=1, keepdims=True)
    dw_ref[...] = lax.dot_general(
        p, rel_ref[...], (((1,), (0,)), ((), ())),
        preferred_element_type=jnp.float32, precision=lax.Precision.HIGHEST)
    nrm = jnp.sqrt(jnp.sum(att * att, axis=1, keepdims=True))
    nt = att / jnp.maximum(nrm, 1e-12)
    sim = lax.dot_general(
        nt, nt, (((1,), (1,)), ((), ())),
        preferred_element_type=jnp.float32, precision=lax.Precision.HIGHEST)
    sc = jnp.exp(sim / 0.2)
    rows = jnp.sum(sc, axis=1)
    nf = att.shape[0]
    ii = lax.broadcasted_iota(jnp.int32, (nf, nf), 0)
    jj = lax.broadcasted_iota(jnp.int32, (nf, nf), 1)
    diag = jnp.sum(jnp.where(ii == jj, sc, 0.0), axis=1)
    cor_ref[...] = (-jnp.sum(jnp.log(diag) - jnp.log(rows))).reshape(1, 1)


def _tc_prep(att, rel):
    nf, nr = att.shape
    return pl.pallas_call(
        _prep_body,
        out_shape=[
            jax.ShapeDtypeStruct((nf, D), jnp.float32),
            jax.ShapeDtypeStruct((1, 1), jnp.float32),
        ],
    )(att, rel)


def kernel(user_emb, entity_emb, latent_emb, relation_emb, disen_weight_att,
           interact_values, edge_index, edge_type, interact_row, interact_col):
    headr = edge_index[0].reshape(KG_SUPERS, SUB, C)
    kgidxr = (edge_type * N_ENT + edge_index[1]).reshape(KG_SUPERS, SUB, C)
    colr = interact_col.reshape(U_SUPERS, SUB, C)
    urowr = interact_row.reshape(U_SUPERS, SUB, C)
    uvalr = interact_values.reshape(U_SUPERS, SUB, C)

    tabL, tabR = _tc_tables(entity_emb, relation_emb)
    s0, s1, u0, u1, cnt = _sc_kernel(True)(
        tabL, tabR, entity_emb[:, :HD], entity_emb[:, HD:],
        kgidxr, headr, colr, urowr, uvalr)
    cnt2d = cnt.reshape(N_ENT_P, 1)
    e1, eres1 = _tc_entity(s0, s1, cnt2d, entity_emb)
    dw, cor = _tc_prep(disen_weight_att, relation_emb)
    u1n, ures1 = _tc_user(u0, u1, user_emb, latent_emb, dw, user_emb)

    tabL2, tabR2 = _tc_tables(e1, relation_emb)
    s0b, s1b, u0b, u1b, _cnt2 = _sc_kernel(False)(
        tabL2, tabR2, e1[:, :HD], e1[:, HD:],
        kgidxr, headr, colr, urowr, uvalr)
    e2, eres2 = _tc_entity(s0b, s1b, cnt2d, eres1)
    u2n, ures2 = _tc_user(u0b, u1b, u1n, latent_emb, dw, ures1)

    return (eres2, ures2, cor.reshape(()))
